# baseline (speedup 1.0000x reference)
import functools

import jax
import jax.numpy as jnp
from jax.experimental import pallas as pl
from jax.experimental.pallas import tpu as pltpu

_CT = 512
_DEPTH = 4


def _router_body(x_hbm, w_ref, b_ref, topi_ref, topw_ref, mu_ref,
                 xbuf, sems, *, n_tokens):
    n_chunks = n_tokens // _CT
    e_dim = w_ref.shape[1]

    def copy_in(c, slot):
        return pltpu.make_async_copy(
            x_hbm.at[pl.ds(c * _CT, _CT), :], xbuf.at[slot], sems.at[slot]
        )

    for c in range(min(_DEPTH, n_chunks)):
        copy_in(c, c).start()

    mu_acc = jnp.zeros((e_dim, 1), jnp.float32)

    for c in range(n_chunks):
        slot = c % _DEPTH
        copy_in(c, slot).wait()

        logits = (
            jnp.dot(xbuf[slot], w_ref[...], preferred_element_type=jnp.float32)
            + b_ref[...]
        )

        nxt = c + _DEPTH
        if nxt < n_chunks:
            copy_in(nxt, slot).start()

        lt = logits.T
        iota = jax.lax.broadcasted_iota(jnp.int32, lt.shape, 0)

        m = jnp.max(lt, axis=0, keepdims=True)
        i1 = jnp.min(jnp.where(lt == m, iota, e_dim), axis=0, keepdims=True)
        masked = jnp.where(iota == i1, -jnp.inf, lt)
        m2 = jnp.max(masked, axis=0, keepdims=True)
        i2 = jnp.min(
            jnp.where(masked == m2, iota, e_dim), axis=0, keepdims=True
        )

        ex = jnp.exp(lt - m)
        s = jnp.sum(ex, axis=0, keepdims=True)
        r = 1.0 / s
        topw_ref[pl.ds(c * _CT, _CT), :] = jnp.concatenate(
            [r, jnp.exp(m2 - m) * r], axis=0
        ).T
        topi_ref[pl.ds(c * _CT, _CT), :] = jnp.concatenate(
            [i1, i2], axis=0
        ).T

        mu_acc = mu_acc + jnp.sum(ex * r, axis=1, keepdims=True)

    mu_ref[...] = mu_acc * (1.0 / n_tokens)


def kernel(x, W, b):
    t, d = x.shape
    e = W.shape[1]

    b2 = b.reshape(1, e)

    topi, topw, mu = pl.pallas_call(
        functools.partial(_router_body, n_tokens=t),
        in_specs=[
            pl.BlockSpec(memory_space=pl.ANY),
            pl.BlockSpec((d, e), lambda: (0, 0)),
            pl.BlockSpec((1, e), lambda: (0, 0)),
        ],
        out_specs=[
            pl.BlockSpec((t, 2), lambda: (0, 0)),
            pl.BlockSpec((t, 2), lambda: (0, 0)),
            pl.BlockSpec((e, 1), lambda: (0, 0)),
        ],
        out_shape=[
            jax.ShapeDtypeStruct((t, 2), jnp.int32),
            jax.ShapeDtypeStruct((t, 2), jnp.float32),
            jax.ShapeDtypeStruct((e, 1), jnp.float32),
        ],
        scratch_shapes=[
            pltpu.VMEM((_DEPTH, _CT, d), jnp.float32),
            pltpu.SemaphoreType.DMA((_DEPTH,)),
        ],
    )(x, W, b2)

    return (topi, topw, mu.reshape(e))
